# baseline (speedup 1.0000x reference)
import jax
import jax.numpy as jnp
from jax.experimental import pallas as pl

_N = 32768
_R = 256
_C = 128
_K = 64
_BIG = 1 << 30
_NEG = float("-inf")


def _merge(va, ia, vb, ib):
    take_a = (va > vb) | ((va == vb) & (ia < ib))
    return jnp.where(take_a, va, vb), jnp.where(take_a, ia, ib)


def _body(x_ref, out_ref):
    x0 = x_ref[:]
    lin = (jax.lax.broadcasted_iota(jnp.int32, (_R, _C), 0) * _C
           + jax.lax.broadcasted_iota(jnp.int32, (_R, _C), 1))
    jcol = jax.lax.broadcasted_iota(jnp.int32, (1, _C), 1)

    a, b = x0[:128, :], x0[128:, :]
    ia, ib = lin[:128, :], lin[128:, :]
    sel = a >= b
    l1v = jnp.where(sel, a, b)
    l1i = jnp.where(sel, ia, ib)
    w1v = jnp.where(sel, b, a)
    w1i = jnp.where(sel, ib, ia)

    def levels(l1v, l1i):
        l2v, l2i = _merge(l1v[:64], l1i[:64], l1v[64:], l1i[64:])
        l3v, l3i = _merge(l2v[:32], l2i[:32], l2v[32:], l2i[32:])
        l4v, l4i = _merge(l3v[:16], l3i[:16], l3v[16:], l3i[16:])
        l5v, l5i = _merge(l4v[:8], l4i[:8], l4v[8:], l4i[8:])
        return l5v, l5i

    l5v, l5i = levels(l1v, l1i)

    def step(i, carry):
        l1v, l1i, w1v, w1i, l5v, l5i, vals, idxs, cs, s = carry
        m = jnp.max(l5v)
        j = jnp.min(jnp.where(l5v == m, l5i, jnp.int32(_BIG)))
        s = s + m
        selc = jcol == i
        vals = jnp.where(selc, m, vals)
        idxs = jnp.where(selc, j, idxs)
        cs = jnp.where(selc, s, cs)
        hit = l1i == j
        l1v = jnp.where(hit, w1v, l1v)
        l1i = jnp.where(hit, w1i, l1i)
        w1v = jnp.where(hit, jnp.float32(_NEG), w1v)
        w1i = jnp.where(hit, jnp.int32(-1), w1i)
        l5v, l5i = levels(l1v, l1i)
        return l1v, l1i, w1v, w1i, l5v, l5i, vals, idxs, cs, s

    carry = (l1v, l1i, w1v, w1i, l5v, l5i,
             jnp.zeros((1, _C), jnp.float32), jnp.zeros((1, _C), jnp.int32),
             jnp.zeros((1, _C), jnp.float32), jnp.float32(0.0))
    carry = jax.lax.fori_loop(0, _K, step, carry)
    vals, idxs, cs = carry[6], carry[7], carry[8]

    kvec = (jcol + 1).astype(jnp.float32)
    support = ((vals - (cs - 1.0) / kvec) > 0.0) & (jcol < _K)
    kz = jnp.sum(support.astype(jnp.int32))
    cs_at = jnp.sum(jnp.where(jcol == kz - 1, cs, 0.0))
    tau = (cs_at - 1.0) / kz.astype(jnp.float32)

    row0 = x0[0:1, :]
    avail = row0 <= tau

    def fstep(t, carry):
        fill, cprev = carry
        cand = jnp.where(avail & (jcol > cprev), jcol, jnp.int32(_BIG))
        c = jnp.min(cand)
        fsel = jcol == (kz + t)
        fill = jnp.where(fsel, c, fill)
        return fill, c

    fill, _ = jax.lax.fori_loop(
        0, _K, fstep, (jnp.zeros((1, _C), jnp.int32), jnp.int32(-1)))

    out = jnp.where(jcol < kz, idxs, fill)
    out_ref[:] = jnp.broadcast_to(out, (8, _C))


def kernel(scores):
    x = scores.reshape(_R, _C)
    out = pl.pallas_call(
        _body,
        out_shape=jax.ShapeDtypeStruct((8, _C), jnp.int32),
    )(x)
    return out[0, :_K]


# speedup vs baseline: 1.0031x; 1.0031x over previous
import jax
import jax.numpy as jnp
from jax.experimental import pallas as pl

_N = 32768
_R = 256
_C = 128
_K = 64
_BIG = 1 << 30
_NEG = float("-inf")


def _merge(va, ia, vb, ib):
    take_a = (va > vb) | ((va == vb) & (ia < ib))
    return jnp.where(take_a, va, vb), jnp.where(take_a, ia, ib)


def _body(x_ref, out_ref):
    x0 = x_ref[:]
    lin = (jax.lax.broadcasted_iota(jnp.int32, (_R, _C), 0) * _C
           + jax.lax.broadcasted_iota(jnp.int32, (_R, _C), 1))
    jcol = jax.lax.broadcasted_iota(jnp.int32, (1, _C), 1)

    a, b = x0[:128, :], x0[128:, :]
    ia, ib = lin[:128, :], lin[128:, :]
    sel = a >= b
    l1v = jnp.where(sel, a, b)
    l1i = jnp.where(sel, ia, ib)
    w1v = jnp.where(sel, b, a)
    w1i = jnp.where(sel, ib, ia)

    def levels(l1v, l1i):
        l2v, l2i = _merge(l1v[:64], l1i[:64], l1v[64:], l1i[64:])
        l3v, l3i = _merge(l2v[:32], l2i[:32], l2v[32:], l2i[32:])
        l4v, l4i = _merge(l3v[:16], l3i[:16], l3v[16:], l3i[16:])
        l5v, l5i = _merge(l4v[:8], l4i[:8], l4v[8:], l4i[8:])
        return l5v, l5i

    l5v, l5i = levels(l1v, l1i)

    def step(i, carry):
        l1v, l1i, w1v, w1i, l5v, l5i, vals, idxs, cs, s = carry
        m = jnp.max(l5v, axis=(0, 1), keepdims=True)
        j = jnp.min(jnp.where(l5v == m, l5i, jnp.int32(_BIG)),
                    axis=(0, 1), keepdims=True)
        s = s + m
        selc = jcol == i
        vals = jnp.where(selc, m, vals)
        idxs = jnp.where(selc, j, idxs)
        cs = jnp.where(selc, s, cs)
        hit = l1i == j
        l1v = jnp.where(hit, w1v, l1v)
        l1i = jnp.where(hit, w1i, l1i)
        w1v = jnp.where(hit, jnp.float32(_NEG), w1v)
        w1i = jnp.where(hit, jnp.int32(-1), w1i)
        l5v, l5i = levels(l1v, l1i)
        return l1v, l1i, w1v, w1i, l5v, l5i, vals, idxs, cs, s

    carry = (l1v, l1i, w1v, w1i, l5v, l5i,
             jnp.zeros((1, _C), jnp.float32), jnp.zeros((1, _C), jnp.int32),
             jnp.zeros((1, _C), jnp.float32), jnp.zeros((1, 1), jnp.float32))
    carry = jax.lax.fori_loop(0, _K, step, carry)
    vals, idxs, cs = carry[6], carry[7], carry[8]

    kvec = (jcol + 1).astype(jnp.float32)
    support = ((vals - (cs - 1.0) / kvec) > 0.0) & (jcol < _K)
    kz = jnp.sum(support.astype(jnp.int32), axis=(0, 1), keepdims=True)
    cs_at = jnp.sum(jnp.where(jcol == kz - 1, cs, 0.0),
                    axis=(0, 1), keepdims=True)
    tau = (cs_at - 1.0) / kz.astype(jnp.float32)

    row0 = x0[0:1, :]
    avail = row0 <= tau

    def fstep(t, carry):
        fill, cprev = carry
        cand = jnp.where(avail & (jcol > cprev), jcol, jnp.int32(_BIG))
        c = jnp.min(cand, axis=(0, 1), keepdims=True)
        fsel = jcol == (kz + t)
        fill = jnp.where(fsel, c, fill)
        return fill, c

    fill, _ = jax.lax.fori_loop(
        0, _K, fstep,
        (jnp.zeros((1, _C), jnp.int32), jnp.full((1, 1), -1, jnp.int32)))

    out = jnp.where(jcol < kz, idxs, fill)
    out_ref[:] = jnp.broadcast_to(out, (8, _C))


def kernel(scores):
    x = scores.reshape(_R, _C)
    out = pl.pallas_call(
        _body,
        out_shape=jax.ShapeDtypeStruct((8, _C), jnp.int32),
    )(x)
    return out[0, :_K]


# speedup vs baseline: 1.0139x; 1.0108x over previous
import jax
import jax.numpy as jnp
from jax.experimental import pallas as pl

_N = 32768
_R = 256
_C = 128
_K = 64
_BIG = 1 << 30
_NEG = float("-inf")


def _merge(va, ia, vb, ib):
    take_a = (va > vb) | ((va == vb) & (ia < ib))
    return jnp.where(take_a, va, vb), jnp.where(take_a, ia, ib)


def _body(x_ref, out_ref):
    x0 = x_ref[:]
    lin = (jax.lax.broadcasted_iota(jnp.int32, (_R, _C), 0) * _C
           + jax.lax.broadcasted_iota(jnp.int32, (_R, _C), 1))
    jcol = jax.lax.broadcasted_iota(jnp.int32, (1, _C), 1)

    a, b = x0[:128, :], x0[128:, :]
    ia, ib = lin[:128, :], lin[128:, :]
    sel = a >= b
    l1v = jnp.where(sel, a, b)
    l1i = jnp.where(sel, ia, ib)
    w1v = jnp.where(sel, b, a)
    w1i = jnp.where(sel, ib, ia)

    def levels(l1v, l1i):
        l2v, l2i = _merge(l1v[:64], l1i[:64], l1v[64:], l1i[64:])
        l3v, l3i = _merge(l2v[:32], l2i[:32], l2v[32:], l2i[32:])
        l4v, l4i = _merge(l3v[:16], l3i[:16], l3v[16:], l3i[16:])
        l5v, l5i = _merge(l4v[:8], l4i[:8], l4v[8:], l4i[8:])
        return l5v, l5i

    l5v, l5i = levels(l1v, l1i)

    def step(i, carry):
        l1v, l1i, w1v, w1i, l5v, l5i, vals, idxs, cs, s = carry
        m = jnp.max(l5v, axis=(0, 1), keepdims=True)
        j = jnp.min(jnp.where(l5v == m, l5i, jnp.int32(_BIG)),
                    axis=(0, 1), keepdims=True)
        s = s + m
        selc = jcol == i
        vals = jnp.where(selc, m, vals)
        idxs = jnp.where(selc, j, idxs)
        cs = jnp.where(selc, s, cs)
        hit = l1i == j
        l1v = jnp.where(hit, w1v, l1v)
        l1i = jnp.where(hit, w1i, l1i)
        w1v = jnp.where(hit, jnp.float32(_NEG), w1v)
        w1i = jnp.where(hit, jnp.int32(-1), w1i)
        l5v, l5i = levels(l1v, l1i)
        return l1v, l1i, w1v, w1i, l5v, l5i, vals, idxs, cs, s

    carry = (l1v, l1i, w1v, w1i, l5v, l5i,
             jnp.zeros((1, _C), jnp.float32), jnp.zeros((1, _C), jnp.int32),
             jnp.zeros((1, _C), jnp.float32), jnp.zeros((1, 1), jnp.float32))
    for _i in range(_K):
        carry = step(_i, carry)
    vals, idxs, cs = carry[6], carry[7], carry[8]

    kvec = (jcol + 1).astype(jnp.float32)
    support = ((vals - (cs - 1.0) / kvec) > 0.0) & (jcol < _K)
    kz = jnp.sum(support.astype(jnp.int32), axis=(0, 1), keepdims=True)
    cs_at = jnp.sum(jnp.where(jcol == kz - 1, cs, 0.0),
                    axis=(0, 1), keepdims=True)
    tau = (cs_at - 1.0) / kz.astype(jnp.float32)

    row0 = x0[0:1, :]
    avail = row0 <= tau

    def fstep(t, carry):
        fill, cprev = carry
        cand = jnp.where(avail & (jcol > cprev), jcol, jnp.int32(_BIG))
        c = jnp.min(cand, axis=(0, 1), keepdims=True)
        fsel = jcol == (kz + t)
        fill = jnp.where(fsel, c, fill)
        return fill, c

    fcarry = (jnp.zeros((1, _C), jnp.int32), jnp.full((1, 1), -1, jnp.int32))
    for _t in range(_K):
        fcarry = fstep(_t, fcarry)
    fill, _ = fcarry

    out = jnp.where(jcol < kz, idxs, fill)
    out_ref[:] = jnp.broadcast_to(out, (8, _C))


def kernel(scores):
    x = scores.reshape(_R, _C)
    out = pl.pallas_call(
        _body,
        out_shape=jax.ShapeDtypeStruct((8, _C), jnp.int32),
    )(x)
    return out[0, :_K]


# speedup vs baseline: 4.0706x; 4.0149x over previous
import jax
import jax.numpy as jnp
from jax.experimental import pallas as pl

_N = 32768
_R = 256
_C = 128
_K = 64


def _swap(x, d, rows):
    parts = []
    for s in range(0, rows, 2 * d):
        parts.append(x[s + d:s + 2 * d])
        parts.append(x[s:s + d])
    return jnp.concatenate(parts, axis=0)


def _gt(va, ia, vb, ib):
    return (va > vb) | ((va == vb) & (ia < ib))


def _cex(v, i, d, k, rows, riota):
    vp = _swap(v, d, rows)
    ip = _swap(i, d, rows)
    ge = _gt(v, i, vp, ip)
    take_max = ((riota & d) == 0) == ((riota & k) == 0)
    takex = ge == take_max
    return jnp.where(takex, v, vp), jnp.where(takex, i, ip)


def _body(x_ref, out_ref):
    v = x_ref[:]
    i = (jax.lax.broadcasted_iota(jnp.int32, (_R, _C), 0) * _C
         + jax.lax.broadcasted_iota(jnp.int32, (_R, _C), 1))
    riota = jax.lax.broadcasted_iota(jnp.int32, (_R, _C), 0)

    k = 2
    while k <= _R:
        d = k // 2
        while d >= 1:
            v, i = _cex(v, i, d, k, _R, riota)
            d //= 2
        k *= 2

    y, yi = v[:_K], i[:_K]

    r64 = jax.lax.broadcasted_iota(jnp.int32, (_K, _C), 0)
    w = _C // 2
    while w >= 1:
        a, ai = y[:, :w], yi[:, :w]
        b, bi = y[:, w:2 * w], yi[:, w:2 * w]
        for d in (32, 16, 8, 4, 2, 1):
            b = _swap(b, d, _K)
            bi = _swap(bi, d, _K)
        ge = _gt(a, ai, b, bi)
        y = jnp.where(ge, a, b)
        yi = jnp.where(ge, ai, bi)
        riota2 = r64[:, :w]
        for d in (32, 16, 8, 4, 2, 1):
            vp = _swap(y, d, _K)
            ip = _swap(yi, d, _K)
            ge = _gt(y, yi, vp, ip)
            take_max = (riota2 & d) == 0
            takex = ge == take_max
            y = jnp.where(takex, y, vp)
            yi = jnp.where(takex, yi, ip)
        w //= 2

    r6 = jax.lax.broadcasted_iota(jnp.int32, (_K, _K), 0)
    c6 = jax.lax.broadcasted_iota(jnp.int32, (_K, _K), 1)
    vb = jnp.broadcast_to(y, (_K, _K))
    ib = jnp.broadcast_to(yi, (_K, _K))
    vals = jnp.sum(jnp.where(r6 == c6, vb, 0.0), axis=0, keepdims=True)
    idxs = jnp.sum(jnp.where(r6 == c6, ib, 0), axis=0, keepdims=True)
    cs = jnp.sum(jnp.where(r6 <= c6, vb, 0.0), axis=0, keepdims=True)

    j64 = jax.lax.broadcasted_iota(jnp.int32, (1, _K), 1)
    kvec = (j64 + 1).astype(jnp.float32)
    support = (vals - (cs - 1.0) / kvec) > 0.0
    kz = jnp.sum(support.astype(jnp.int32), axis=(0, 1), keepdims=True)
    cs_at = jnp.sum(jnp.where(j64 == kz - 1, cs, 0.0),
                    axis=(0, 1), keepdims=True)
    tau = (cs_at - 1.0) / kz.astype(jnp.float32)

    row0 = x_ref[0:1, :]
    avail = row0 <= tau
    rc = jax.lax.broadcasted_iota(jnp.int32, (_C, _C), 0)
    cc = jax.lax.broadcasted_iota(jnp.int32, (_C, _C), 1)
    ab = jnp.broadcast_to(avail, (_C, _C))
    pc_col = jnp.sum(jnp.where(ab & (cc <= rc), 1, 0), axis=1, keepdims=True)
    av_col = jnp.sum(jnp.where(ab & (cc == rc), 1, 0), axis=1, keepdims=True)
    tgt_col = kz + pc_col - 1
    mfill = (av_col > 0) & (tgt_col == cc)
    fill = jnp.sum(jnp.where(mfill, rc, 0), axis=0, keepdims=True)

    jcol = jax.lax.broadcasted_iota(jnp.int32, (1, _C), 1)
    idx128 = jnp.concatenate([idxs, jnp.zeros((1, _C - _K), jnp.int32)],
                             axis=1)
    out = jnp.where(jcol < kz, idx128, fill)
    out_ref[:] = jnp.broadcast_to(out, (8, _C))


def kernel(scores):
    x = scores.reshape(_R, _C)
    out = pl.pallas_call(
        _body,
        out_shape=jax.ShapeDtypeStruct((8, _C), jnp.int32),
    )(x)
    return out[0, :_K]


# speedup vs baseline: 4.3547x; 1.0698x over previous
import jax
import jax.numpy as jnp
from jax.experimental import pallas as pl

_N = 32768
_R = 256
_C = 128
_K = 64


def _swap(x, d, rows, riota):
    if d >= 8:
        parts = []
        for s in range(0, rows, 2 * d):
            parts.append(x[s + d:s + 2 * d])
            parts.append(x[s:s + d])
        return jnp.concatenate(parts, axis=0)
    up = jnp.concatenate([x[d:], x[:d]], axis=0)
    dn = jnp.concatenate([x[rows - d:], x[:rows - d]], axis=0)
    return jnp.where((riota & d) == 0, up, dn)


def _gt(va, ia, vb, ib):
    return (va > vb) | ((va == vb) & (ia < ib))


def _cex(v, i, d, k, rows, riota):
    vp = _swap(v, d, rows, riota)
    ip = _swap(i, d, rows, riota)
    ge = _gt(v, i, vp, ip)
    take_max = ((riota & d) == 0) == ((riota & k) == 0)
    takex = ge == take_max
    return jnp.where(takex, v, vp), jnp.where(takex, i, ip)


def _body(x_ref, out_ref):
    v = x_ref[:]
    i = (jax.lax.broadcasted_iota(jnp.int32, (_R, _C), 0) * _C
         + jax.lax.broadcasted_iota(jnp.int32, (_R, _C), 1))
    riota = jax.lax.broadcasted_iota(jnp.int32, (_R, _C), 0)

    k = 2
    while k <= _R:
        d = k // 2
        while d >= 1:
            v, i = _cex(v, i, d, k, _R, riota)
            d //= 2
        k *= 2

    y, yi = v[:_K], i[:_K]

    r64 = jax.lax.broadcasted_iota(jnp.int32, (_K, _C), 0)
    w = _C // 2
    while w >= 1:
        a, ai = y[:, :w], yi[:, :w]
        b, bi = y[:, w:2 * w], yi[:, w:2 * w]
        riota2 = r64[:, :w]
        for d in (32, 16, 8, 4, 2, 1):
            b = _swap(b, d, _K, riota2)
            bi = _swap(bi, d, _K, riota2)
        ge = _gt(a, ai, b, bi)
        y = jnp.where(ge, a, b)
        yi = jnp.where(ge, ai, bi)
        for d in (32, 16, 8, 4, 2, 1):
            vp = _swap(y, d, _K, riota2)
            ip = _swap(yi, d, _K, riota2)
            ge = _gt(y, yi, vp, ip)
            take_max = (riota2 & d) == 0
            takex = ge == take_max
            y = jnp.where(takex, y, vp)
            yi = jnp.where(takex, yi, ip)
        w //= 2

    r6 = jax.lax.broadcasted_iota(jnp.int32, (_K, _K), 0)
    c6 = jax.lax.broadcasted_iota(jnp.int32, (_K, _K), 1)
    vb = jnp.broadcast_to(y, (_K, _K))
    ib = jnp.broadcast_to(yi, (_K, _K))
    vals = jnp.sum(jnp.where(r6 == c6, vb, 0.0), axis=0, keepdims=True)
    idxs = jnp.sum(jnp.where(r6 == c6, ib, 0), axis=0, keepdims=True)
    cs = jnp.sum(jnp.where(r6 <= c6, vb, 0.0), axis=0, keepdims=True)

    j64 = jax.lax.broadcasted_iota(jnp.int32, (1, _K), 1)
    kvec = (j64 + 1).astype(jnp.float32)
    support = (vals - (cs - 1.0) / kvec) > 0.0
    kz = jnp.sum(support.astype(jnp.int32), axis=(0, 1), keepdims=True)
    cs_at = jnp.sum(jnp.where(j64 == kz - 1, cs, 0.0),
                    axis=(0, 1), keepdims=True)
    tau = (cs_at - 1.0) / kz.astype(jnp.float32)

    row0 = x_ref[0:1, :]
    avail = row0 <= tau
    rc = jax.lax.broadcasted_iota(jnp.int32, (_C, _C), 0)
    cc = jax.lax.broadcasted_iota(jnp.int32, (_C, _C), 1)
    ab = jnp.broadcast_to(avail, (_C, _C))
    pc_col = jnp.sum(jnp.where(ab & (cc <= rc), 1, 0), axis=1, keepdims=True)
    av_col = jnp.sum(jnp.where(ab & (cc == rc), 1, 0), axis=1, keepdims=True)
    tgt_col = kz + pc_col - 1
    mfill = (av_col > 0) & (tgt_col == cc)
    fill = jnp.sum(jnp.where(mfill, rc, 0), axis=0, keepdims=True)

    jcol = jax.lax.broadcasted_iota(jnp.int32, (1, _C), 1)
    idx128 = jnp.concatenate([idxs, jnp.zeros((1, _C - _K), jnp.int32)],
                             axis=1)
    out = jnp.where(jcol < kz, idx128, fill)
    out_ref[:] = jnp.broadcast_to(out, (8, _C))


def kernel(scores):
    x = scores.reshape(_R, _C)
    out = pl.pallas_call(
        _body,
        out_shape=jax.ShapeDtypeStruct((8, _C), jnp.int32),
    )(x)
    return out[0, :_K]
